# baseline (speedup 1.0000x reference)
import jax
import jax.numpy as jnp
from jax.experimental import pallas as pl
from jax.experimental.pallas import tpu as pltpu

_E = 8
_K = 2
_H = 768
_TBLK = 2048


def _router_block(w_ref, b_ref, hs_ref, scores_ref, idx_ref):
    hs = hs_ref[...]
    w = w_ref[...]
    logits = jax.lax.dot_general(
        w, hs, (((1,), (1,)), ((), ())), preferred_element_type=jnp.float32)
    logits = logits + b_ref[...]

    e_iota = jax.lax.broadcasted_iota(jnp.int32, logits.shape, 0)
    m1 = jnp.max(logits, axis=0, keepdims=True)
    i1 = jnp.min(jnp.where(logits == m1, e_iota, _E), axis=0, keepdims=True)
    masked = jnp.where(e_iota == i1, -jnp.inf, logits)
    m2 = jnp.max(masked, axis=0, keepdims=True)
    i2 = jnp.min(jnp.where(masked == m2, e_iota, _E), axis=0, keepdims=True)

    s = jnp.exp(m2 - m1)
    r = 1.0 / (1.0 + s)
    scores_t = (jnp.where(e_iota == i1, r, 0.0)
                + jnp.where(e_iota == i2, s * r, 0.0))
    scores_ref[...] = scores_t
    idx_ref[...] = jnp.concatenate([i1, i2], axis=0)[None]


@jax.jit
def kernel(hidden_states, router_weight, router_bias):
    t = hidden_states.shape[0]
    nblk = t // _TBLK
    scores_p, idx_p = pl.pallas_call(
        _router_block,
        grid=(nblk,),
        in_specs=[
            pl.BlockSpec((_E, _H), lambda i: (0, 0)),
            pl.BlockSpec((_E, 1), lambda i: (0, 0)),
            pl.BlockSpec((_TBLK, _H), lambda i: (i, 0)),
        ],
        out_specs=[
            pl.BlockSpec((_E, _TBLK), lambda i: (i, 0)),
            pl.BlockSpec((1, _K, _TBLK), lambda i: (i, 0, 0)),
        ],
        out_shape=[
            jax.ShapeDtypeStruct((nblk * _E, _TBLK), jnp.float32),
            jax.ShapeDtypeStruct((nblk, _K, _TBLK), jnp.int32),
        ],
        compiler_params=pltpu.CompilerParams(
            dimension_semantics=("parallel",)),
    )(router_weight, router_bias.reshape(_E, 1), hidden_states)
    scores = scores_p.reshape(nblk, _E, _TBLK).transpose(0, 2, 1).reshape(t, _E)
    idx = idx_p.transpose(0, 2, 1).reshape(t, _K)
    return scores, idx


# speedup vs baseline: 1.0389x; 1.0389x over previous
import functools

import jax
import jax.numpy as jnp
from jax.experimental import pallas as pl
from jax.experimental.pallas import tpu as pltpu

_E = 8
_K = 2
_H = 768
_T = 32768
_CH = 1024
_NCH = _T // _CH
_DEPTH = 4


def _router_body(w_ref, b_ref, hs_any, scores_ref, idx_ref, bufs, sems):
    w = w_ref[...]
    b = b_ref[...]

    def start(i, slot):
        pltpu.make_async_copy(
            hs_any.at[pl.ds(i * _CH, _CH), :], bufs.at[slot], sems.at[slot]
        ).start()

    def wait(i, slot):
        pltpu.make_async_copy(
            hs_any.at[pl.ds(i * _CH, _CH), :], bufs.at[slot], sems.at[slot]
        ).wait()

    for d in range(_DEPTH):
        start(d, d)

    def step(i, carry):
        slot = jax.lax.rem(i, _DEPTH)
        wait(i, slot)
        hs = bufs[slot]
        logits = jax.lax.dot_general(
            w, hs, (((1,), (1,)), ((), ())),
            preferred_element_type=jnp.float32) + b

        e_iota = jax.lax.broadcasted_iota(jnp.int32, logits.shape, 0)
        m1 = jnp.max(logits, axis=0, keepdims=True)
        i1 = jnp.min(jnp.where(logits == m1, e_iota, _E), axis=0, keepdims=True)
        masked = jnp.where(e_iota == i1, -jnp.inf, logits)
        m2 = jnp.max(masked, axis=0, keepdims=True)
        i2 = jnp.min(jnp.where(masked == m2, e_iota, _E), axis=0, keepdims=True)

        s = jnp.exp(m2 - m1)
        r = 1.0 / (1.0 + s)
        scores_t = (jnp.where(e_iota == i1, r, 0.0)
                    + jnp.where(e_iota == i2, s * r, 0.0))
        scores_ref[:, pl.ds(i * _CH, _CH)] = scores_t
        idx_ref[:, pl.ds(i * _CH, _CH)] = jnp.concatenate([i1, i2], axis=0)

        @pl.when(i + _DEPTH < _NCH)
        def _():
            start(i + _DEPTH, slot)

        return carry

    jax.lax.fori_loop(0, _NCH, step, 0)


@jax.jit
def kernel(hidden_states, router_weight, router_bias):
    t = hidden_states.shape[0]
    scores_p, idx_p = pl.pallas_call(
        _router_body,
        in_specs=[
            pl.BlockSpec(memory_space=pltpu.VMEM),
            pl.BlockSpec(memory_space=pltpu.VMEM),
            pl.BlockSpec(memory_space=pl.ANY),
        ],
        out_specs=[
            pl.BlockSpec(memory_space=pltpu.VMEM),
            pl.BlockSpec(memory_space=pltpu.VMEM),
        ],
        out_shape=[
            jax.ShapeDtypeStruct((_E, t), jnp.float32),
            jax.ShapeDtypeStruct((_K, t), jnp.int32),
        ],
        scratch_shapes=[
            pltpu.VMEM((_DEPTH, _CH, _H), jnp.float32),
            pltpu.SemaphoreType.DMA((_DEPTH,)),
        ],
    )(router_weight, router_bias.reshape(_E, 1), hidden_states)
    return scores_p.T, idx_p.T
